# Initial kernel scaffold; baseline (speedup 1.0000x reference)
import jax
import jax.numpy as jnp
from jax.experimental import pallas as pl


def kernel(x, emb_table, lin_w, lin_bias, clf_W, clf_b):
    raise NotImplementedError("write your pallas kernel here")


# speedup vs baseline: 1.2773x; 1.2773x over previous
import functools

import jax
import jax.numpy as jnp
from jax import lax
from jax.experimental import pallas as pl
from jax.experimental.pallas import tpu as pltpu
from jax.experimental.pallas import tpu_sc as plsc

B, F, NFEAT, NEMB = 16384, 26, 1000000, 16
NC, NS, L = 2, 16, 16
NW = NC * NS
SPW = B // NW
CH = 128
NCHUNK = SPW // CH
CI = CH * F


@functools.partial(
    pl.kernel,
    out_type=jax.ShapeDtypeStruct((B,), jnp.float32),
    mesh=plsc.VectorSubcoreMesh(core_axis_name="c", subcore_axis_name="s"),
    compiler_params=pltpu.CompilerParams(
        needs_layout_passes=False, use_tc_tiling_on_sc=False),
    scratch_types=[
        pltpu.VMEM((CI,), jnp.int32),
        pltpu.VMEM((CI, L), jnp.float32),
        pltpu.VMEM((CI + L,), jnp.float32),
        pltpu.VMEM((CH,), jnp.float32),
        pltpu.VMEM((4, L), jnp.float32),
        pltpu.SemaphoreType.DMA,
        pltpu.SemaphoreType.DMA,
    ],
)
def _fm_sc(x_hbm, emb_hbm, linw_hbm, wv_hbm, out_hbm,
           idx_v, rows_v, linv_v, out_v, wv_v, sem_e, sem_l):
    wid = lax.axis_index("s") * NC + lax.axis_index("c")
    base = wid * SPW
    pltpu.sync_copy(wv_hbm, wv_v)
    wvec = wv_v[0, :]
    wsum_vec = wv_v[1, :]
    cvec = wv_v[2, :]
    lanes = lax.iota(jnp.int32, L)
    tail_mask = lanes < (F - L)

    def chunk_body(c, carry):
        cb = base + c * CH
        pltpu.sync_copy(x_hbm.at[pl.ds(cb * F, CI)], idx_v)
        cp_e = pltpu.async_copy(emb_hbm.at[idx_v], rows_v, sem_e)
        cp_l = pltpu.async_copy(linw_hbm.at[idx_v],
                                linv_v.at[pl.ds(0, CI)], sem_l)
        cp_e.wait()
        cp_l.wait()

        def group_body(g, carry2):
            acc = jnp.zeros((L,), jnp.float32)
            for j in range(L):
                rb = (g * L + j) * F
                v0 = rows_v[rb, :]
                S = v0
                Q = v0 * v0
                for f in range(1, F):
                    v = rows_v[rb + f, :]
                    S = S + v
                    Q = Q + v * v
                fm = 0.5 * (S * S - Q)
                la = linv_v[pl.ds(rb, L)]
                lb = jnp.where(tail_mask, linv_v[pl.ds(rb + L, L)], 0.0)
                t = fm * wvec + (la + lb) * wsum_vec + cvec
                acc = jnp.where(lanes == j, jnp.sum(t), acc)
            out_v[pl.ds(g * L, L)] = acc
            return carry2

        lax.fori_loop(0, CH // L, group_body, 0)
        pltpu.sync_copy(out_v, out_hbm.at[pl.ds(cb, CH)])
        return carry

    lax.fori_loop(0, NCHUNK, chunk_body, 0)


def kernel(x, emb_table, lin_w, lin_bias, clf_W, clf_b):
    wvec = clf_W[:, 0].astype(jnp.float32)
    wsum = jnp.sum(wvec)
    const = lin_bias * wsum + clf_b[0]
    wv = jnp.stack([
        wvec,
        jnp.full((L,), 1.0, jnp.float32) * wsum,
        jnp.full((L,), 1.0, jnp.float32) * (const / L),
        jnp.zeros((L,), jnp.float32),
    ])
    out = _fm_sc(x.reshape(-1), emb_table, lin_w, wv)
    return out.reshape(B, 1)
